# baseline (speedup 1.0000x reference)
import functools

import jax
import jax.numpy as jnp
from jax import lax
from jax.experimental import pallas as pl
from jax.experimental.pallas import tpu as pltpu
from jax.experimental.pallas import tpu_sc as plsc

NC = 2
NS = 16
L = 16


def _sc_mesh():
    return plsc.VectorSubcoreMesh(
        core_axis_name="c", subcore_axis_name="s", num_cores=NC, num_subcores=NS
    )


def _make_count_kernel(CH, HRF):
    CHW = CH // (NC * NS)

    def body(src_hbm, dst_hbm, cnt_out, m_out, src_v, dst_v, chist, mhist):
        cid = lax.axis_index("c")
        sid = lax.axis_index("s")
        w = cid * NS + sid

        def zhist(i, _):
            z = jnp.zeros((L,), jnp.float32)
            chist[pl.ds(i * L, L)] = z
            mhist[pl.ds(i * L, L)] = z
            return 0
        lax.fori_loop(0, HRF // L, zhist, 0)

        pltpu.sync_copy(src_hbm.at[pl.ds(w * CHW, CHW)], src_v)
        pltpu.sync_copy(dst_hbm.at[pl.ds(w * CHW, CHW)], dst_v)

        ones = jnp.ones((L,), jnp.float32)

        def edge_row(j, _):
            for k in range(8):
                sl = pl.ds(k * L, L)
                dv = dst_v[j, sl]
                sv = src_v[j, sl]
                plsc.addupdate_scatter(chist, [dv], ones)
                mval = jnp.where(dv == 0, 1.0, 0.0).astype(jnp.float32)
                plsc.addupdate_scatter(mhist, [sv], mval)
            return 0
        lax.fori_loop(0, CHW, edge_row, 0)

        pltpu.sync_copy(chist, cnt_out.at[w])
        pltpu.sync_copy(mhist, m_out.at[w])

    out_t = (jax.ShapeDtypeStruct((NC * NS, HRF), jnp.float32),
             jax.ShapeDtypeStruct((NC * NS, HRF), jnp.float32))
    return pl.kernel(
        body, out_type=out_t, mesh=_sc_mesh(),
        compiler_params=pltpu.CompilerParams(needs_layout_passes=False),
        scratch_types=[
            pltpu.VMEM((CHW, 128), jnp.int32),
            pltpu.VMEM((CHW, 128), jnp.int32),
            pltpu.VMEM((HRF,), jnp.float32),
            pltpu.VMEM((HRF,), jnp.float32),
        ],
    )


def _make_prop_kernel(CH, NROW, Dh):
    CHW = CH // NS
    AR = ((NROW + 1 + 127) // 128) * 128
    ZPT = AR // NS
    ZB = 16
    SB = 4
    NSUP = CHW // SB
    assert CHW % SB == 0 and NSUP % 2 == 0

    def body(u_hbm, gsrc_hbm, dst_hbm, y_hbm, sb0, sb1, db0, db1, r0, r1,
             zbuf, acc, rs0, rs1, ss0, ss1, sd0, sd1):
        srcb = (sb0, sb1)
        dstb = (db0, db1)
        rows = (r0, r1)
        rsem = (rs0, rs1)
        ssem = (ss0, ss1)
        dsem = (sd0, sd1)
        cid = lax.axis_index("c")
        sid = lax.axis_index("s")
        tbase = sid * CHW

        def fire_idx(s, p):
            pltpu.async_copy(
                gsrc_hbm.at[cid, pl.ds(tbase + s * SB, SB)], srcb[p], ssem[p])
            pltpu.async_copy(
                dst_hbm.at[pl.ds(tbase + s * SB, SB)], dstb[p], dsem[p])

        def wait_idx(s, p):
            pltpu.make_async_copy(
                gsrc_hbm.at[cid, pl.ds(tbase + s * SB, SB)], srcb[p],
                ssem[p]).wait()
            pltpu.make_async_copy(
                dst_hbm.at[pl.ds(tbase + s * SB, SB)], dstb[p],
                dsem[p]).wait()

        def fire_gather(q, row, b):
            pltpu.async_copy(u_hbm.at[srcb[q].at[row]], rows[b], rsem[b])

        def wait_gather(q, row, b):
            pltpu.make_async_copy(
                u_hbm.at[srcb[q].at[row]], rows[b], rsem[b]).wait()

        for r in range(ZB):
            for k in range(Dh // L):
                zbuf[r, pl.ds(k * L, L)] = jnp.zeros((L,), jnp.float32)
        base = sid * ZPT
        done = 0
        while done < ZPT:
            n = min(ZB, ZPT - done)
            pltpu.sync_copy(zbuf.at[pl.ds(0, n)], acc.at[pl.ds(base + done, n)])
            done += n
        plsc.subcore_barrier()

        fire_idx(0, 0)
        fire_idx(1, 1)
        wait_idx(0, 0)
        fire_gather(0, 0, 0)
        fire_gather(0, 1, 1)

        def outer(i, _):
            for p in range(2):
                s = i * 2 + p
                for jj in range(SB):
                    j = s * SB + jj
                    b = jj % 2
                    wait_gather(p, jj, b)
                    pltpu.sync_copy(rows[b], acc.at[dstb[p].at[jj]],
                                    add=True)
                    if jj == SB - 2:
                        @pl.when(s + 1 < NSUP)
                        def _():
                            wait_idx(s + 1, 1 - p)
                    nj = jj + 2
                    q, row = (p, nj) if nj < SB else (1 - p, nj - SB)

                    @pl.when(j + 2 < CHW)
                    def _():
                        fire_gather(q, row, b)
                @pl.when(s + 2 < NSUP)
                def _():
                    fire_idx(s + 2, p)
            return 0
        lax.fori_loop(0, NSUP // 2, outer, 0)
        plsc.subcore_barrier()

        pltpu.sync_copy(acc.at[pl.ds(sid * ZPT, ZPT)],
                        y_hbm.at[cid, pl.ds(sid * ZPT, ZPT)])

    return pl.kernel(
        body,
        out_type=jax.ShapeDtypeStruct((NC, AR, Dh), jnp.float32),
        mesh=_sc_mesh(),
        compiler_params=pltpu.CompilerParams(
            needs_layout_passes=False, use_tc_tiling_on_sc=False),
        scratch_types=[
            pltpu.VMEM((SB, 128), jnp.int32),
            pltpu.VMEM((SB, 128), jnp.int32),
            pltpu.VMEM((SB, 128), jnp.int32),
            pltpu.VMEM((SB, 128), jnp.int32),
            pltpu.VMEM((128, Dh), jnp.float32),
            pltpu.VMEM((128, Dh), jnp.float32),
            pltpu.VMEM((ZB, Dh), jnp.float32),
            pltpu.VMEM_SHARED((AR, Dh), jnp.float32),
            pltpu.SemaphoreType.DMA,
            pltpu.SemaphoreType.DMA,
            pltpu.SemaphoreType.DMA,
            pltpu.SemaphoreType.DMA,
            pltpu.SemaphoreType.DMA,
            pltpu.SemaphoreType.DMA,
        ],
    )


def _make_prop_bf16(CH, NROW):
    Dh = 256
    CHW = CH // NS // 2
    AR = ((NROW + 1 + 127) // 128) * 128
    ZPT = AR // NS
    ZB = 16
    SB = 4
    NSUP = CHW // SB
    assert CHW % SB == 0 and NSUP % 2 == 0

    def body(u_hbm, src_hbm, dst_hbm, y_hbm, sb0, sb1, db0, db1, r0, r1,
             zbuf, acc, rs0, rs1, ss0, ss1, sd0, sd1):
        srcb = (sb0, sb1)
        dstb = (db0, db1)
        rows = (r0, r1)
        rsem = (rs0, rs1)
        ssem = (ss0, ss1)
        dsem = (sd0, sd1)
        cid = lax.axis_index("c")
        sid = lax.axis_index("s")
        tbase = cid * (CH // 2) + sid * CHW

        def fire_idx(s, p):
            pltpu.async_copy(
                src_hbm.at[pl.ds(tbase + s * SB, SB)], srcb[p], ssem[p])
            pltpu.async_copy(
                dst_hbm.at[pl.ds(tbase + s * SB, SB)], dstb[p], dsem[p])

        def wait_idx(s, p):
            pltpu.make_async_copy(
                src_hbm.at[pl.ds(tbase + s * SB, SB)], srcb[p],
                ssem[p]).wait()
            pltpu.make_async_copy(
                dst_hbm.at[pl.ds(tbase + s * SB, SB)], dstb[p],
                dsem[p]).wait()

        def fire_gather(q, row, b):
            pltpu.async_copy(u_hbm.at[srcb[q].at[row]], rows[b], rsem[b])

        def wait_gather(q, row, b):
            pltpu.make_async_copy(
                u_hbm.at[srcb[q].at[row]], rows[b], rsem[b]).wait()

        zv = jnp.zeros((2 * L,), jnp.bfloat16)
        for r in range(ZB):
            for k in range(Dh // (2 * L)):
                zbuf[r, pl.ds(k * 2 * L, 2 * L)] = zv
        base = sid * ZPT
        done = 0
        while done < ZPT:
            n = min(ZB, ZPT - done)
            pltpu.sync_copy(zbuf.at[pl.ds(0, n)], acc.at[pl.ds(base + done, n)])
            done += n
        plsc.subcore_barrier()

        fire_idx(0, 0)
        fire_idx(1, 1)
        wait_idx(0, 0)
        fire_gather(0, 0, 0)
        fire_gather(0, 1, 1)

        def outer(i, _):
            for p in range(2):
                s = i * 2 + p
                for jj in range(SB):
                    j = s * SB + jj
                    b = jj % 2
                    wait_gather(p, jj, b)
                    pltpu.sync_copy(rows[b], acc.at[dstb[p].at[jj]],
                                    add=True)
                    if jj == SB - 2:
                        @pl.when(s + 1 < NSUP)
                        def _():
                            wait_idx(s + 1, 1 - p)
                    nj = jj + 2
                    q, row = (p, nj) if nj < SB else (1 - p, nj - SB)

                    @pl.when(j + 2 < CHW)
                    def _():
                        fire_gather(q, row, b)
                @pl.when(s + 2 < NSUP)
                def _():
                    fire_idx(s + 2, p)
            return 0
        lax.fori_loop(0, NSUP // 2, outer, 0)
        plsc.subcore_barrier()

        pltpu.sync_copy(acc.at[pl.ds(sid * ZPT, ZPT)],
                        y_hbm.at[cid, pl.ds(sid * ZPT, ZPT)])

    return pl.kernel(
        body,
        out_type=jax.ShapeDtypeStruct((NC, AR, Dh), jnp.bfloat16),
        mesh=_sc_mesh(),
        compiler_params=pltpu.CompilerParams(
            needs_layout_passes=False, use_tc_tiling_on_sc=False),
        scratch_types=[
            pltpu.VMEM((SB, 128), jnp.int32),
            pltpu.VMEM((SB, 128), jnp.int32),
            pltpu.VMEM((SB, 128), jnp.int32),
            pltpu.VMEM((SB, 128), jnp.int32),
            pltpu.VMEM((128, Dh), jnp.bfloat16),
            pltpu.VMEM((128, Dh), jnp.bfloat16),
            pltpu.VMEM((ZB, Dh), jnp.bfloat16),
            pltpu.VMEM_SHARED((AR, Dh), jnp.bfloat16),
            pltpu.SemaphoreType.DMA,
            pltpu.SemaphoreType.DMA,
            pltpu.SemaphoreType.DMA,
            pltpu.SemaphoreType.DMA,
            pltpu.SemaphoreType.DMA,
            pltpu.SemaphoreType.DMA,
        ],
    )


def _stats_body(cnt_ref, m_ref, dinv_ref, cful_ref):
    counts = jnp.sum(cnt_ref[...], axis=0, keepdims=True)
    dinv = lax.rsqrt(counts + 1.0)
    m = jnp.sum(m_ref[...], axis=0, keepdims=True)
    dinv0 = dinv[0, 0]
    cc = lax.broadcasted_iota(jnp.int32, dinv.shape, 1)
    self0 = jnp.where(cc == 0, dinv0 * dinv0, 0.0)
    dinv_ref[...] = dinv
    cful_ref[...] = m * dinv * dinv0 + self0


def _scale_body(x_ref, dinv_ref, o_ref):
    o_ref[...] = (x_ref[...] * dinv_ref[...]).astype(jnp.bfloat16)


def _layer1_body(y_ref, u_ref, dinv_ref, w1_ref, b1_ref, w2_ref, o_ref):
    y = (y_ref[0].astype(jnp.float32) + y_ref[1].astype(jnp.float32)
         + u_ref[...].astype(jnp.float32))
    dinv = dinv_ref[...]
    g = y * dinv
    h = jnp.maximum(jnp.dot(g, w1_ref[...],
                            preferred_element_type=jnp.float32) + b1_ref[...], 0.0)
    t = jnp.dot(h, w2_ref[...], preferred_element_type=jnp.float32)
    o_ref[...] = (t * dinv).astype(jnp.bfloat16)


def _layer2_body(y_ref, u_ref, dinv_ref, b2_ref, w3_ref, o_ref):
    y = (y_ref[0].astype(jnp.float32) + y_ref[1].astype(jnp.float32)
         + u_ref[...].astype(jnp.float32))
    dinv = dinv_ref[...]
    g = y * dinv
    h = jnp.maximum(g + b2_ref[...], 0.0)
    t = jnp.dot(h, w3_ref[...], preferred_element_type=jnp.float32)
    o_ref[...] = t * dinv


def _final_body(y_ref, u_ref, dinv_ref, cful_ref, b3_ref, w4_ref, b4_ref,
                wl_ref, bl_ref, o_ref, sacc):
    i = pl.program_id(0)

    @pl.when(i == 0)
    def _():
        sacc[...] = jnp.zeros_like(sacc)

    y = jnp.concatenate([y_ref[0], y_ref[1]], axis=1)
    g = (y + u_ref[...]) * dinv_ref[...]
    h3 = jnp.maximum(g + b3_ref[...], 0.0)
    sacc[...] += jnp.sum(h3 * cful_ref[...], axis=0, keepdims=True)

    @pl.when(i == pl.num_programs(0) - 1)
    def _():
        r = jnp.dot(sacc[...], w4_ref[...],
                    preferred_element_type=jnp.float32) + b4_ref[...]
        o_ref[...] = jnp.dot(r, wl_ref[...],
                             preferred_element_type=jnp.float32) + bl_ref[...]


def kernel(x, edge_index, W1, b1, W2, b2, W3, b3, W4, b4, Wl, bl):
    N, D_IN = x.shape
    E = edge_index.shape[1]
    BN = 400
    NB = N // BN

    ei = edge_index.astype(jnp.int32)
    EP = ((E + 4095) // 4096) * 4096
    CH = EP // 128
    src = jnp.concatenate([ei[0], jnp.zeros((EP - E,), jnp.int32)])
    dst = jnp.concatenate([ei[1], jnp.full((EP - E,), N, jnp.int32)])
    src2d = src.reshape(CH, 128)
    dst2d = dst.reshape(CH, 128)
    gsrc3d = jnp.stack([src2d * 2, src2d * 2 + 1])

    HRF = ((N + 1 + 127) // 128) * 128
    cnt_p, m_p = _make_count_kernel(CH, HRF)(src2d, dst2d)

    dinv2d, cful2d = pl.pallas_call(
        _stats_body,
        out_shape=(jax.ShapeDtypeStruct((1, HRF), jnp.float32),
                   jax.ShapeDtypeStruct((1, HRF), jnp.float32)),
    )(cnt_p, m_p)
    dinv = dinv2d.reshape(-1)[:N].reshape(N, 1)
    cful = cful2d.reshape(-1)[:N].reshape(N, 1)

    row_spec = pl.BlockSpec((BN, D_IN), lambda i: (i, 0))
    dv_spec = pl.BlockSpec((BN, 1), lambda i: (i, 0))

    u1 = pl.pallas_call(
        _scale_body, grid=(NB,),
        in_specs=[row_spec, dv_spec],
        out_specs=row_spec,
        out_shape=jax.ShapeDtypeStruct((N, D_IN), jnp.bfloat16),
    )(x, dinv)

    prop256 = _make_prop_bf16(CH, N)
    prop64 = _make_prop_kernel(CH, N, 32)

    y1 = prop256(u1, src2d, dst2d)[:, :N]

    y_spec = pl.BlockSpec((NC, BN, 256), lambda i: (0, i, 0))
    full = lambda a, b: pl.BlockSpec((a, b), lambda i: (0, 0))

    u2 = pl.pallas_call(
        _layer1_body, grid=(NB,),
        in_specs=[y_spec, row_spec, dv_spec, full(256, 1024), full(1, 1024),
                  full(1024, 256)],
        out_specs=pl.BlockSpec((BN, 256), lambda i: (i, 0)),
        out_shape=jax.ShapeDtypeStruct((N, 256), jnp.bfloat16),
    )(y1, u1, dinv, W1, b1.reshape(1, -1), W2)

    y2 = prop256(u2, src2d, dst2d)[:, :N]

    u3 = pl.pallas_call(
        _layer2_body, grid=(NB,),
        in_specs=[y_spec, pl.BlockSpec((BN, 256), lambda i: (i, 0)), dv_spec,
                  full(1, 256), full(256, 64)],
        out_specs=pl.BlockSpec((BN, 64), lambda i: (i, 0)),
        out_shape=jax.ShapeDtypeStruct((N, 64), jnp.float32),
    )(y2, u2, dinv, b2.reshape(1, -1), W3)

    y3 = prop64(u3.reshape(2 * N, 32), gsrc3d, dst2d)[:, :N]

    out = pl.pallas_call(
        _final_body, grid=(NB,),
        in_specs=[pl.BlockSpec((NC, BN, 32), lambda i: (0, i, 0)),
                  pl.BlockSpec((BN, 64), lambda i: (i, 0)), dv_spec, dv_spec,
                  full(1, 64), full(64, 16), full(1, 16), full(16, 3),
                  full(1, 3)],
        out_specs=pl.BlockSpec((1, 3), lambda i: (0, 0)),
        out_shape=jax.ShapeDtypeStruct((1, 3), jnp.float32),
        scratch_shapes=[pltpu.VMEM((1, 64), jnp.float32)],
    )(y3, u3, dinv, cful, b3.reshape(1, -1), W4, b4.reshape(1, -1), Wl,
      bl.reshape(1, -1))

    return out


# speedup vs baseline: 1.0152x; 1.0152x over previous
import functools

import jax
import jax.numpy as jnp
from jax import lax
from jax.experimental import pallas as pl
from jax.experimental.pallas import tpu as pltpu
from jax.experimental.pallas import tpu_sc as plsc

NC = 2
NS = 16
L = 16


def _sc_mesh():
    return plsc.VectorSubcoreMesh(
        core_axis_name="c", subcore_axis_name="s", num_cores=NC, num_subcores=NS
    )


def _make_count_kernel(CH, HRF):
    CHW = CH // (NC * NS)

    def body(src_hbm, dst_hbm, cnt_out, m_out, src_v, dst_v, chist, mhist):
        cid = lax.axis_index("c")
        sid = lax.axis_index("s")
        w = cid * NS + sid

        def zhist(i, _):
            z = jnp.zeros((L,), jnp.float32)
            chist[pl.ds(i * L, L)] = z
            mhist[pl.ds(i * L, L)] = z
            return 0
        lax.fori_loop(0, HRF // L, zhist, 0)

        pltpu.sync_copy(src_hbm.at[pl.ds(w * CHW, CHW)], src_v)
        pltpu.sync_copy(dst_hbm.at[pl.ds(w * CHW, CHW)], dst_v)

        ones = jnp.ones((L,), jnp.float32)

        def edge_row(j, _):
            for k in range(8):
                sl = pl.ds(k * L, L)
                dv = dst_v[j, sl]
                sv = src_v[j, sl]
                plsc.addupdate_scatter(chist, [dv], ones)
                mval = jnp.where(dv == 0, 1.0, 0.0).astype(jnp.float32)
                plsc.addupdate_scatter(mhist, [sv], mval)
            return 0
        lax.fori_loop(0, CHW, edge_row, 0)

        pltpu.sync_copy(chist, cnt_out.at[w])
        pltpu.sync_copy(mhist, m_out.at[w])

    out_t = (jax.ShapeDtypeStruct((NC * NS, HRF), jnp.float32),
             jax.ShapeDtypeStruct((NC * NS, HRF), jnp.float32))
    return pl.kernel(
        body, out_type=out_t, mesh=_sc_mesh(),
        compiler_params=pltpu.CompilerParams(needs_layout_passes=False),
        scratch_types=[
            pltpu.VMEM((CHW, 128), jnp.int32),
            pltpu.VMEM((CHW, 128), jnp.int32),
            pltpu.VMEM((HRF,), jnp.float32),
            pltpu.VMEM((HRF,), jnp.float32),
        ],
    )


def _make_prop_kernel(CH, NROW, Dh):
    CHW = CH // NS
    AR = ((NROW + 1 + 127) // 128) * 128
    ZPT = AR // NS
    ZB = 16
    SB = 4
    NSUP = CHW // SB
    assert CHW % SB == 0 and NSUP % 2 == 0

    def body(u_hbm, gsrc_hbm, dst_hbm, y_hbm, sb0, sb1, db0, db1, r0, r1,
             zbuf, acc, rs0, rs1, ss0, ss1, sd0, sd1):
        srcb = (sb0, sb1)
        dstb = (db0, db1)
        rows = (r0, r1)
        rsem = (rs0, rs1)
        ssem = (ss0, ss1)
        dsem = (sd0, sd1)
        cid = lax.axis_index("c")
        sid = lax.axis_index("s")
        tbase = sid * CHW

        def fire_idx(s, p):
            pltpu.async_copy(
                gsrc_hbm.at[cid, pl.ds(tbase + s * SB, SB)], srcb[p], ssem[p])
            pltpu.async_copy(
                dst_hbm.at[pl.ds(tbase + s * SB, SB)], dstb[p], dsem[p])

        def wait_idx(s, p):
            pltpu.make_async_copy(
                gsrc_hbm.at[cid, pl.ds(tbase + s * SB, SB)], srcb[p],
                ssem[p]).wait()
            pltpu.make_async_copy(
                dst_hbm.at[pl.ds(tbase + s * SB, SB)], dstb[p],
                dsem[p]).wait()

        def fire_gather(q, row, b):
            pltpu.async_copy(u_hbm.at[srcb[q].at[row]], rows[b], rsem[b])

        def wait_gather(q, row, b):
            pltpu.make_async_copy(
                u_hbm.at[srcb[q].at[row]], rows[b], rsem[b]).wait()

        for r in range(ZB):
            for k in range(Dh // L):
                zbuf[r, pl.ds(k * L, L)] = jnp.zeros((L,), jnp.float32)
        base = sid * ZPT
        done = 0
        while done < ZPT:
            n = min(ZB, ZPT - done)
            pltpu.sync_copy(zbuf.at[pl.ds(0, n)], acc.at[pl.ds(base + done, n)])
            done += n
        plsc.subcore_barrier()

        fire_idx(0, 0)
        fire_idx(1, 1)
        wait_idx(0, 0)
        fire_gather(0, 0, 0)
        fire_gather(0, 1, 1)

        def outer(i, _):
            for p in range(2):
                s = i * 2 + p
                for jj in range(SB):
                    j = s * SB + jj
                    b = jj % 2
                    wait_gather(p, jj, b)
                    pltpu.sync_copy(rows[b], acc.at[dstb[p].at[jj]],
                                    add=True)
                    if jj == SB - 2:
                        @pl.when(s + 1 < NSUP)
                        def _():
                            wait_idx(s + 1, 1 - p)
                    nj = jj + 2
                    q, row = (p, nj) if nj < SB else (1 - p, nj - SB)

                    @pl.when(j + 2 < CHW)
                    def _():
                        fire_gather(q, row, b)
                @pl.when(s + 2 < NSUP)
                def _():
                    fire_idx(s + 2, p)
            return 0
        lax.fori_loop(0, NSUP // 2, outer, 0)
        plsc.subcore_barrier()

        pltpu.sync_copy(acc.at[pl.ds(sid * ZPT, ZPT)],
                        y_hbm.at[cid, pl.ds(sid * ZPT, ZPT)])

    return pl.kernel(
        body,
        out_type=jax.ShapeDtypeStruct((NC, AR, Dh), jnp.float32),
        mesh=_sc_mesh(),
        compiler_params=pltpu.CompilerParams(
            needs_layout_passes=False, use_tc_tiling_on_sc=False),
        scratch_types=[
            pltpu.VMEM((SB, 128), jnp.int32),
            pltpu.VMEM((SB, 128), jnp.int32),
            pltpu.VMEM((SB, 128), jnp.int32),
            pltpu.VMEM((SB, 128), jnp.int32),
            pltpu.VMEM((128, Dh), jnp.float32),
            pltpu.VMEM((128, Dh), jnp.float32),
            pltpu.VMEM((ZB, Dh), jnp.float32),
            pltpu.VMEM_SHARED((AR, Dh), jnp.float32),
            pltpu.SemaphoreType.DMA,
            pltpu.SemaphoreType.DMA,
            pltpu.SemaphoreType.DMA,
            pltpu.SemaphoreType.DMA,
            pltpu.SemaphoreType.DMA,
            pltpu.SemaphoreType.DMA,
        ],
    )


def _make_prop_bf16(CH, NROW):
    Dh = 256
    CHW = CH // NS // 2
    AR = ((NROW + 1 + 127) // 128) * 128
    ZPT = AR // NS
    ZB = 16
    SB = 4
    NSUP = CHW // SB
    assert CHW % SB == 0 and NSUP % 2 == 0

    def body(u_hbm, src_hbm, dst_hbm, y_hbm, sb0, sb1, db0, db1, r0, r1,
             zbuf, acc, rs0, rs1, ss0, ss1, sd0, sd1):
        srcb = (sb0, sb1)
        dstb = (db0, db1)
        rows = (r0, r1)
        rsem = (rs0, rs1)
        ssem = (ss0, ss1)
        dsem = (sd0, sd1)
        cid = lax.axis_index("c")
        sid = lax.axis_index("s")
        tbase = cid * (CH // 2) + sid * CHW

        def fire_idx(s, p):
            pltpu.async_copy(
                src_hbm.at[pl.ds(tbase + s * SB, SB)], srcb[p], ssem[p])
            pltpu.async_copy(
                dst_hbm.at[pl.ds(tbase + s * SB, SB)], dstb[p], dsem[p])

        def wait_idx(s, p):
            pltpu.make_async_copy(
                src_hbm.at[pl.ds(tbase + s * SB, SB)], srcb[p],
                ssem[p]).wait()
            pltpu.make_async_copy(
                dst_hbm.at[pl.ds(tbase + s * SB, SB)], dstb[p],
                dsem[p]).wait()

        def fire_gather(q, row, b):
            pltpu.async_copy(u_hbm.at[srcb[q].at[row]], rows[b], rsem[b])

        def wait_gather(q, row, b):
            pltpu.make_async_copy(
                u_hbm.at[srcb[q].at[row]], rows[b], rsem[b]).wait()

        zv = jnp.zeros((2 * L,), jnp.bfloat16)
        for r in range(ZB):
            for k in range(Dh // (2 * L)):
                zbuf[r, pl.ds(k * 2 * L, 2 * L)] = zv
        base = sid * ZPT
        done = 0
        while done < ZPT:
            n = min(ZB, ZPT - done)
            pltpu.sync_copy(zbuf.at[pl.ds(0, n)], acc.at[pl.ds(base + done, n)])
            done += n
        plsc.subcore_barrier()

        fire_idx(0, 0)
        fire_idx(1, 1)
        wait_idx(0, 0)
        fire_gather(0, 0, 0)
        fire_gather(0, 1, 1)

        def outer(i, _):
            for p in range(2):
                s = i * 2 + p
                for jj in range(SB):
                    j = s * SB + jj
                    b = jj % 2
                    wait_gather(p, jj, b)
                    pltpu.sync_copy(rows[b], acc.at[dstb[p].at[jj]],
                                    add=True)
                    if jj == SB - 2:
                        @pl.when(s + 1 < NSUP)
                        def _():
                            wait_idx(s + 1, 1 - p)
                    nj = jj + 2
                    q, row = (p, nj) if nj < SB else (1 - p, nj - SB)

                    @pl.when(j + 2 < CHW)
                    def _():
                        fire_gather(q, row, b)
                @pl.when(s + 2 < NSUP)
                def _():
                    fire_idx(s + 2, p)
            return 0
        lax.fori_loop(0, NSUP // 2, outer, 0)
        plsc.subcore_barrier()

        pltpu.sync_copy(acc.at[pl.ds(sid * ZPT, ZPT)],
                        y_hbm.at[cid, pl.ds(sid * ZPT, ZPT)])

    return pl.kernel(
        body,
        out_type=jax.ShapeDtypeStruct((NC, AR, Dh), jnp.bfloat16),
        mesh=_sc_mesh(),
        compiler_params=pltpu.CompilerParams(
            needs_layout_passes=False, use_tc_tiling_on_sc=False),
        scratch_types=[
            pltpu.VMEM((SB, 128), jnp.int32),
            pltpu.VMEM((SB, 128), jnp.int32),
            pltpu.VMEM((SB, 128), jnp.int32),
            pltpu.VMEM((SB, 128), jnp.int32),
            pltpu.VMEM((128, Dh), jnp.bfloat16),
            pltpu.VMEM((128, Dh), jnp.bfloat16),
            pltpu.VMEM((ZB, Dh), jnp.bfloat16),
            pltpu.VMEM_SHARED((AR, Dh), jnp.bfloat16),
            pltpu.SemaphoreType.DMA,
            pltpu.SemaphoreType.DMA,
            pltpu.SemaphoreType.DMA,
            pltpu.SemaphoreType.DMA,
            pltpu.SemaphoreType.DMA,
            pltpu.SemaphoreType.DMA,
        ],
    )


def _stats_body(cnt_ref, m_ref, dinv_ref, cful_ref):
    counts = jnp.sum(cnt_ref[...], axis=0, keepdims=True)
    dinv = lax.rsqrt(counts + 1.0)
    m = jnp.sum(m_ref[...], axis=0, keepdims=True)
    dinv0 = dinv[0, 0]
    cc = lax.broadcasted_iota(jnp.int32, dinv.shape, 1)
    self0 = jnp.where(cc == 0, dinv0 * dinv0, 0.0)
    dinv_ref[...] = dinv
    cful_ref[...] = m * dinv * dinv0 + self0


def _scale_body(x_ref, dinv_ref, o_ref):
    o_ref[...] = (x_ref[...] * dinv_ref[...]).astype(jnp.bfloat16)


def _layer1_body(y_ref, u_ref, dinv_ref, w1_ref, b1_ref, w2_ref, o_ref):
    y = (y_ref[0].astype(jnp.float32) + y_ref[1].astype(jnp.float32)
         + u_ref[...].astype(jnp.float32))
    dinv = dinv_ref[...]
    g = y * dinv
    h = jnp.maximum(jnp.dot(g, w1_ref[...],
                            preferred_element_type=jnp.float32) + b1_ref[...], 0.0)
    t = jnp.dot(h, w2_ref[...], preferred_element_type=jnp.float32)
    o_ref[...] = (t * dinv).astype(jnp.bfloat16)


def _layer2_body(y_ref, u_ref, dinv_ref, b2_ref, w3_ref, o_ref):
    y = (y_ref[0].astype(jnp.float32) + y_ref[1].astype(jnp.float32)
         + u_ref[...].astype(jnp.float32))
    dinv = dinv_ref[...]
    g = y * dinv
    h = jnp.maximum(g + b2_ref[...], 0.0)
    t = jnp.dot(h, w3_ref[...], preferred_element_type=jnp.float32)
    o_ref[...] = t * dinv


def _final_body(y_ref, u_ref, dinv_ref, cful_ref, b3_ref, w4_ref, b4_ref,
                wl_ref, bl_ref, o_ref, sacc):
    i = pl.program_id(0)

    @pl.when(i == 0)
    def _():
        sacc[...] = jnp.zeros_like(sacc)

    y = jnp.concatenate([y_ref[0], y_ref[1]], axis=1)
    g = (y + u_ref[...]) * dinv_ref[...]
    h3 = jnp.maximum(g + b3_ref[...], 0.0)
    sacc[...] += jnp.sum(h3 * cful_ref[...], axis=0, keepdims=True)

    @pl.when(i == pl.num_programs(0) - 1)
    def _():
        r = jnp.dot(sacc[...], w4_ref[...],
                    preferred_element_type=jnp.float32) + b4_ref[...]
        o_ref[...] = jnp.dot(r, wl_ref[...],
                             preferred_element_type=jnp.float32) + bl_ref[...]


def kernel(x, edge_index, W1, b1, W2, b2, W3, b3, W4, b4, Wl, bl):
    N, D_IN = x.shape
    E = edge_index.shape[1]
    BN = 400
    NB = N // BN

    ei = edge_index.astype(jnp.int32)
    EP = ((E + 4095) // 4096) * 4096
    CH = EP // 128
    src = jnp.concatenate([ei[0], jnp.zeros((EP - E,), jnp.int32)])
    pad_dst = N + (jnp.arange(EP - E, dtype=jnp.int32) % 112)
    dst = jnp.concatenate([ei[1], pad_dst])
    src2d = src.reshape(CH, 128)
    dst2d = dst.reshape(CH, 128)
    gsrc3d = jnp.stack([src2d * 2, src2d * 2 + 1])

    HRF = ((N + 1 + 127) // 128) * 128
    cnt_p, m_p = _make_count_kernel(CH, HRF)(src2d, dst2d)

    dinv2d, cful2d = pl.pallas_call(
        _stats_body,
        out_shape=(jax.ShapeDtypeStruct((1, HRF), jnp.float32),
                   jax.ShapeDtypeStruct((1, HRF), jnp.float32)),
    )(cnt_p, m_p)
    dinv = dinv2d.reshape(-1)[:N].reshape(N, 1)
    cful = cful2d.reshape(-1)[:N].reshape(N, 1)

    row_spec = pl.BlockSpec((BN, D_IN), lambda i: (i, 0))
    dv_spec = pl.BlockSpec((BN, 1), lambda i: (i, 0))

    u1 = pl.pallas_call(
        _scale_body, grid=(NB,),
        in_specs=[row_spec, dv_spec],
        out_specs=row_spec,
        out_shape=jax.ShapeDtypeStruct((N, D_IN), jnp.bfloat16),
    )(x, dinv)

    prop256 = _make_prop_bf16(CH, N)
    prop64 = _make_prop_kernel(CH, N, 32)

    y1 = prop256(u1, src2d, dst2d)[:, :N]

    y_spec = pl.BlockSpec((NC, BN, 256), lambda i: (0, i, 0))
    full = lambda a, b: pl.BlockSpec((a, b), lambda i: (0, 0))

    u2 = pl.pallas_call(
        _layer1_body, grid=(NB,),
        in_specs=[y_spec, row_spec, dv_spec, full(256, 1024), full(1, 1024),
                  full(1024, 256)],
        out_specs=pl.BlockSpec((BN, 256), lambda i: (i, 0)),
        out_shape=jax.ShapeDtypeStruct((N, 256), jnp.bfloat16),
    )(y1, u1, dinv, W1, b1.reshape(1, -1), W2)

    y2 = prop256(u2, src2d, dst2d)[:, :N]

    u3 = pl.pallas_call(
        _layer2_body, grid=(NB,),
        in_specs=[y_spec, pl.BlockSpec((BN, 256), lambda i: (i, 0)), dv_spec,
                  full(1, 256), full(256, 64)],
        out_specs=pl.BlockSpec((BN, 64), lambda i: (i, 0)),
        out_shape=jax.ShapeDtypeStruct((N, 64), jnp.float32),
    )(y2, u2, dinv, b2.reshape(1, -1), W3)

    y3 = prop64(u3.reshape(2 * N, 32), gsrc3d, dst2d)[:, :N]

    out = pl.pallas_call(
        _final_body, grid=(NB,),
        in_specs=[pl.BlockSpec((NC, BN, 32), lambda i: (0, i, 0)),
                  pl.BlockSpec((BN, 64), lambda i: (i, 0)), dv_spec, dv_spec,
                  full(1, 64), full(64, 16), full(1, 16), full(16, 3),
                  full(1, 3)],
        out_specs=pl.BlockSpec((1, 3), lambda i: (0, 0)),
        out_shape=jax.ShapeDtypeStruct((1, 3), jnp.float32),
        scratch_shapes=[pltpu.VMEM((1, 64), jnp.float32)],
    )(y3, u3, dinv, cful, b3.reshape(1, -1), W4, b4.reshape(1, -1), Wl,
      bl.reshape(1, -1))

    return out


# speedup vs baseline: 1.9977x; 1.9677x over previous
import functools

import jax
import jax.numpy as jnp
from jax import lax
from jax.experimental import pallas as pl
from jax.experimental.pallas import tpu as pltpu
from jax.experimental.pallas import tpu_sc as plsc

NC = 2
NS = 16
L = 16


def _sc_mesh():
    return plsc.VectorSubcoreMesh(
        core_axis_name="c", subcore_axis_name="s", num_cores=NC, num_subcores=NS
    )


def _make_count_kernel(CH, HRF):
    CHW = CH // (NC * NS)

    def body(src_hbm, dst_hbm, cnt_out, m_out, src_v, dst_v, chist, mhist):
        cid = lax.axis_index("c")
        sid = lax.axis_index("s")
        w = cid * NS + sid

        def zhist(i, _):
            z = jnp.zeros((L,), jnp.float32)
            chist[pl.ds(i * L, L)] = z
            mhist[pl.ds(i * L, L)] = z
            return 0
        lax.fori_loop(0, HRF // L, zhist, 0)

        pltpu.sync_copy(src_hbm.at[pl.ds(w * CHW, CHW)], src_v)
        pltpu.sync_copy(dst_hbm.at[pl.ds(w * CHW, CHW)], dst_v)

        ones = jnp.ones((L,), jnp.float32)

        def edge_row(j, _):
            for k in range(8):
                sl = pl.ds(k * L, L)
                dv = dst_v[j, sl]
                sv = src_v[j, sl]
                plsc.addupdate_scatter(chist, [dv], ones)
                mval = jnp.where(dv == 0, 1.0, 0.0).astype(jnp.float32)
                plsc.addupdate_scatter(mhist, [sv], mval)
            return 0
        lax.fori_loop(0, CHW, edge_row, 0)

        pltpu.sync_copy(chist, cnt_out.at[w])
        pltpu.sync_copy(mhist, m_out.at[w])

    out_t = (jax.ShapeDtypeStruct((NC * NS, HRF), jnp.float32),
             jax.ShapeDtypeStruct((NC * NS, HRF), jnp.float32))
    return pl.kernel(
        body, out_type=out_t, mesh=_sc_mesh(),
        compiler_params=pltpu.CompilerParams(needs_layout_passes=False),
        scratch_types=[
            pltpu.VMEM((CHW, 128), jnp.int32),
            pltpu.VMEM((CHW, 128), jnp.int32),
            pltpu.VMEM((HRF,), jnp.float32),
            pltpu.VMEM((HRF,), jnp.float32),
        ],
    )


def _make_prop_kernel(CH, NROW, Dh):
    CHW = CH // NS
    AR = ((NROW + 1 + 127) // 128) * 128
    ZPT = AR // NS
    ZB = 16
    SB = 4
    NSUP = CHW // SB
    assert CHW % SB == 0 and NSUP % 2 == 0

    def body(u_hbm, gsrc_hbm, dst_hbm, y_hbm, sb0, sb1, db0, db1, r0, r1,
             zbuf, acc, rs0, rs1, ss0, ss1, sd0, sd1):
        srcb = (sb0, sb1)
        dstb = (db0, db1)
        rows = (r0, r1)
        rsem = (rs0, rs1)
        ssem = (ss0, ss1)
        dsem = (sd0, sd1)
        cid = lax.axis_index("c")
        sid = lax.axis_index("s")
        tbase = sid * CHW

        def fire_idx(s, p):
            pltpu.async_copy(
                gsrc_hbm.at[cid, pl.ds(tbase + s * SB, SB)], srcb[p], ssem[p])
            pltpu.async_copy(
                dst_hbm.at[pl.ds(tbase + s * SB, SB)], dstb[p], dsem[p])

        def wait_idx(s, p):
            pltpu.make_async_copy(
                gsrc_hbm.at[cid, pl.ds(tbase + s * SB, SB)], srcb[p],
                ssem[p]).wait()
            pltpu.make_async_copy(
                dst_hbm.at[pl.ds(tbase + s * SB, SB)], dstb[p],
                dsem[p]).wait()

        def fire_gather(q, row, b):
            pltpu.async_copy(u_hbm.at[srcb[q].at[row]], rows[b], rsem[b])

        def wait_gather(q, row, b):
            pltpu.make_async_copy(
                u_hbm.at[srcb[q].at[row]], rows[b], rsem[b]).wait()

        for r in range(ZB):
            for k in range(Dh // L):
                zbuf[r, pl.ds(k * L, L)] = jnp.zeros((L,), jnp.float32)
        base = sid * ZPT
        done = 0
        while done < ZPT:
            n = min(ZB, ZPT - done)
            pltpu.sync_copy(zbuf.at[pl.ds(0, n)], acc.at[pl.ds(base + done, n)])
            done += n
        plsc.subcore_barrier()

        fire_idx(0, 0)
        fire_idx(1, 1)
        wait_idx(0, 0)
        fire_gather(0, 0, 0)
        fire_gather(0, 1, 1)

        def outer(i, _):
            for p in range(2):
                s = i * 2 + p
                for jj in range(SB):
                    j = s * SB + jj
                    b = jj % 2
                    wait_gather(p, jj, b)
                    pltpu.sync_copy(rows[b], acc.at[dstb[p].at[jj]],
                                    add=True)
                    if jj == SB - 2:
                        @pl.when(s + 1 < NSUP)
                        def _():
                            wait_idx(s + 1, 1 - p)
                    nj = jj + 2
                    q, row = (p, nj) if nj < SB else (1 - p, nj - SB)

                    @pl.when(j + 2 < CHW)
                    def _():
                        fire_gather(q, row, b)
                @pl.when(s + 2 < NSUP)
                def _():
                    fire_idx(s + 2, p)
            return 0
        lax.fori_loop(0, NSUP // 2, outer, 0)
        plsc.subcore_barrier()

        pltpu.sync_copy(acc.at[pl.ds(sid * ZPT, ZPT)],
                        y_hbm.at[cid, pl.ds(sid * ZPT, ZPT)])

    return pl.kernel(
        body,
        out_type=jax.ShapeDtypeStruct((NC, AR, Dh), jnp.float32),
        mesh=_sc_mesh(),
        compiler_params=pltpu.CompilerParams(
            needs_layout_passes=False, use_tc_tiling_on_sc=False),
        scratch_types=[
            pltpu.VMEM((SB, 128), jnp.int32),
            pltpu.VMEM((SB, 128), jnp.int32),
            pltpu.VMEM((SB, 128), jnp.int32),
            pltpu.VMEM((SB, 128), jnp.int32),
            pltpu.VMEM((128, Dh), jnp.float32),
            pltpu.VMEM((128, Dh), jnp.float32),
            pltpu.VMEM((ZB, Dh), jnp.float32),
            pltpu.VMEM_SHARED((AR, Dh), jnp.float32),
            pltpu.SemaphoreType.DMA,
            pltpu.SemaphoreType.DMA,
            pltpu.SemaphoreType.DMA,
            pltpu.SemaphoreType.DMA,
            pltpu.SemaphoreType.DMA,
            pltpu.SemaphoreType.DMA,
        ],
    )


def _make_prop_bf16(CH, NROW):
    Dh = 256
    CHW = CH // NS // 2
    AR = ((NROW + 1 + 127) // 128) * 128
    ZPT = AR // NS
    ZB = 16
    SB = 4
    NSUP = CHW // SB
    assert CHW % SB == 0 and NSUP % 2 == 0

    def body(u_hbm, src_hbm, dst_hbm, y_hbm, sb0, sb1, db0, db1, r0, r1,
             zbuf, acc, rs0, rs1, ss0, ss1, sd0, sd1):
        srcb = (sb0, sb1)
        dstb = (db0, db1)
        rows = (r0, r1)
        rsem = (rs0, rs1)
        ssem = (ss0, ss1)
        dsem = (sd0, sd1)
        cid = lax.axis_index("c")
        sid = lax.axis_index("s")
        tbase = cid * (CH // 2) + sid * CHW

        def fire_idx(s, p):
            pltpu.async_copy(
                src_hbm.at[pl.ds(tbase + s * SB, SB)], srcb[p], ssem[p])
            pltpu.async_copy(
                dst_hbm.at[pl.ds(tbase + s * SB, SB)], dstb[p], dsem[p])

        def wait_idx(s, p):
            pltpu.make_async_copy(
                src_hbm.at[pl.ds(tbase + s * SB, SB)], srcb[p],
                ssem[p]).wait()
            pltpu.make_async_copy(
                dst_hbm.at[pl.ds(tbase + s * SB, SB)], dstb[p],
                dsem[p]).wait()

        def fire_gather(q, row, b):
            pltpu.async_copy(u_hbm.at[srcb[q].at[row]], rows[b], rsem[b])

        def wait_gather(q, row, b):
            pltpu.make_async_copy(
                u_hbm.at[srcb[q].at[row]], rows[b], rsem[b]).wait()

        zv = jnp.zeros((2 * L,), jnp.bfloat16)
        for r in range(ZB):
            for k in range(Dh // (2 * L)):
                zbuf[r, pl.ds(k * 2 * L, 2 * L)] = zv
        base = sid * ZPT
        done = 0
        while done < ZPT:
            n = min(ZB, ZPT - done)
            pltpu.sync_copy(zbuf.at[pl.ds(0, n)], acc.at[pl.ds(base + done, n)])
            done += n
        plsc.subcore_barrier()

        fire_idx(0, 0)
        fire_idx(1, 1)
        wait_idx(0, 0)
        fire_gather(0, 0, 0)
        fire_gather(0, 1, 1)

        def outer(i, _):
            for p in range(2):
                s = i * 2 + p
                for jj in range(SB):
                    j = s * SB + jj
                    b = jj % 2
                    wait_gather(p, jj, b)
                    pltpu.sync_copy(rows[b], acc.at[dstb[p].at[jj]],
                                    add=True)
                    if jj == SB - 2:
                        @pl.when(s + 1 < NSUP)
                        def _():
                            wait_idx(s + 1, 1 - p)
                    nj = jj + 2
                    q, row = (p, nj) if nj < SB else (1 - p, nj - SB)

                    @pl.when(j + 2 < CHW)
                    def _():
                        fire_gather(q, row, b)
                @pl.when(s + 2 < NSUP)
                def _():
                    fire_idx(s + 2, p)
            return 0
        lax.fori_loop(0, NSUP // 2, outer, 0)
        plsc.subcore_barrier()

        pltpu.sync_copy(acc.at[pl.ds(sid * ZPT, ZPT)],
                        y_hbm.at[cid, pl.ds(sid * ZPT, ZPT)])

    return pl.kernel(
        body,
        out_type=jax.ShapeDtypeStruct((NC, AR, Dh), jnp.bfloat16),
        mesh=_sc_mesh(),
        compiler_params=pltpu.CompilerParams(
            needs_layout_passes=False, use_tc_tiling_on_sc=False),
        scratch_types=[
            pltpu.VMEM((SB, 128), jnp.int32),
            pltpu.VMEM((SB, 128), jnp.int32),
            pltpu.VMEM((SB, 128), jnp.int32),
            pltpu.VMEM((SB, 128), jnp.int32),
            pltpu.VMEM((128, Dh), jnp.bfloat16),
            pltpu.VMEM((128, Dh), jnp.bfloat16),
            pltpu.VMEM((ZB, Dh), jnp.bfloat16),
            pltpu.VMEM_SHARED((AR, Dh), jnp.bfloat16),
            pltpu.SemaphoreType.DMA,
            pltpu.SemaphoreType.DMA,
            pltpu.SemaphoreType.DMA,
            pltpu.SemaphoreType.DMA,
            pltpu.SemaphoreType.DMA,
            pltpu.SemaphoreType.DMA,
        ],
    )


def _stats_body(cnt_ref, m_ref, dinv_ref, cful_ref):
    counts = jnp.sum(cnt_ref[...], axis=0, keepdims=True)
    dinv = lax.rsqrt(counts + 1.0)
    m = jnp.sum(m_ref[...], axis=0, keepdims=True)
    dinv0 = dinv[0, 0]
    cc = lax.broadcasted_iota(jnp.int32, dinv.shape, 1)
    self0 = jnp.where(cc == 0, dinv0 * dinv0, 0.0)
    dinv_ref[...] = dinv
    cful_ref[...] = m * dinv * dinv0 + self0


def _scale_body(x_ref, dinv_ref, o_ref):
    o_ref[...] = (x_ref[...] * dinv_ref[...]).astype(jnp.bfloat16)


def _layer1_body(y_ref, u_ref, dinv_ref, w1_ref, b1_ref, w2_ref, o_ref):
    y = (y_ref[0].astype(jnp.float32) + y_ref[1].astype(jnp.float32)
         + u_ref[...].astype(jnp.float32))
    dinv = dinv_ref[...]
    g = y * dinv
    h = jnp.maximum(jnp.dot(g, w1_ref[...],
                            preferred_element_type=jnp.float32) + b1_ref[...], 0.0)
    t = jnp.dot(h, w2_ref[...], preferred_element_type=jnp.float32)
    o_ref[...] = (t * dinv).astype(jnp.bfloat16)


def _layer2_body(y_ref, u_ref, dinv_ref, b2_ref, w3_ref, o_ref):
    y = (y_ref[0].astype(jnp.float32) + y_ref[1].astype(jnp.float32)
         + u_ref[...].astype(jnp.float32))
    dinv = dinv_ref[...]
    g = y * dinv
    h = jnp.maximum(g + b2_ref[...], 0.0)
    t = jnp.dot(h, w3_ref[...], preferred_element_type=jnp.float32)
    o_ref[...] = t * dinv


def _final_body(y_ref, u_ref, dinv_ref, cful_ref, b3_ref, w4_ref, b4_ref,
                wl_ref, bl_ref, o_ref, sacc):
    i = pl.program_id(0)

    @pl.when(i == 0)
    def _():
        sacc[...] = jnp.zeros_like(sacc)

    y = jnp.concatenate([y_ref[0], y_ref[1]], axis=1)
    g = (y + u_ref[...]) * dinv_ref[...]
    h3 = jnp.maximum(g + b3_ref[...], 0.0)
    sacc[...] += jnp.sum(h3 * cful_ref[...], axis=0, keepdims=True)

    @pl.when(i == pl.num_programs(0) - 1)
    def _():
        r = jnp.dot(sacc[...], w4_ref[...],
                    preferred_element_type=jnp.float32) + b4_ref[...]
        o_ref[...] = jnp.dot(r, wl_ref[...],
                             preferred_element_type=jnp.float32) + bl_ref[...]


def kernel(x, edge_index, W1, b1, W2, b2, W3, b3, W4, b4, Wl, bl):
    N, D_IN = x.shape
    E = edge_index.shape[1]
    BN = 400
    NB = N // BN

    ei = edge_index.astype(jnp.int32)
    EP = ((E + 4095) // 4096) * 4096
    CH = EP // 128
    pad_iota = jnp.arange(EP - E, dtype=jnp.int32)
    src = jnp.concatenate([ei[0], pad_iota % N])
    dst = jnp.concatenate([ei[1], N + (pad_iota % 112)])
    src2d = src.reshape(CH, 128)
    dst2d = dst.reshape(CH, 128)
    gsrc3d = jnp.stack([src2d * 2, src2d * 2 + 1])

    HRF = ((N + 1 + 127) // 128) * 128
    cnt_p, m_p = _make_count_kernel(CH, HRF)(src2d, dst2d)

    dinv2d, cful2d = pl.pallas_call(
        _stats_body,
        out_shape=(jax.ShapeDtypeStruct((1, HRF), jnp.float32),
                   jax.ShapeDtypeStruct((1, HRF), jnp.float32)),
    )(cnt_p, m_p)
    dinv = dinv2d.reshape(-1)[:N].reshape(N, 1)
    cful = cful2d.reshape(-1)[:N].reshape(N, 1)

    row_spec = pl.BlockSpec((BN, D_IN), lambda i: (i, 0))
    dv_spec = pl.BlockSpec((BN, 1), lambda i: (i, 0))

    u1 = pl.pallas_call(
        _scale_body, grid=(NB,),
        in_specs=[row_spec, dv_spec],
        out_specs=row_spec,
        out_shape=jax.ShapeDtypeStruct((N, D_IN), jnp.bfloat16),
    )(x, dinv)

    prop256 = _make_prop_bf16(CH, N)
    prop64 = _make_prop_kernel(CH, N, 32)

    y1 = prop256(u1, src2d, dst2d)[:, :N]

    y_spec = pl.BlockSpec((NC, BN, 256), lambda i: (0, i, 0))
    full = lambda a, b: pl.BlockSpec((a, b), lambda i: (0, 0))

    u2 = pl.pallas_call(
        _layer1_body, grid=(NB,),
        in_specs=[y_spec, row_spec, dv_spec, full(256, 1024), full(1, 1024),
                  full(1024, 256)],
        out_specs=pl.BlockSpec((BN, 256), lambda i: (i, 0)),
        out_shape=jax.ShapeDtypeStruct((N, 256), jnp.bfloat16),
    )(y1, u1, dinv, W1, b1.reshape(1, -1), W2)

    y2 = prop256(u2, src2d, dst2d)[:, :N]

    u3 = pl.pallas_call(
        _layer2_body, grid=(NB,),
        in_specs=[y_spec, pl.BlockSpec((BN, 256), lambda i: (i, 0)), dv_spec,
                  full(1, 256), full(256, 64)],
        out_specs=pl.BlockSpec((BN, 64), lambda i: (i, 0)),
        out_shape=jax.ShapeDtypeStruct((N, 64), jnp.float32),
    )(y2, u2, dinv, b2.reshape(1, -1), W3)

    y3 = prop64(u3.reshape(2 * N, 32), gsrc3d, dst2d)[:, :N]

    out = pl.pallas_call(
        _final_body, grid=(NB,),
        in_specs=[pl.BlockSpec((NC, BN, 32), lambda i: (0, i, 0)),
                  pl.BlockSpec((BN, 64), lambda i: (i, 0)), dv_spec, dv_spec,
                  full(1, 64), full(64, 16), full(1, 16), full(16, 3),
                  full(1, 3)],
        out_specs=pl.BlockSpec((1, 3), lambda i: (0, 0)),
        out_shape=jax.ShapeDtypeStruct((1, 3), jnp.float32),
        scratch_shapes=[pltpu.VMEM((1, 64), jnp.float32)],
    )(y3, u3, dinv, cful, b3.reshape(1, -1), W4, b4.reshape(1, -1), Wl,
      bl.reshape(1, -1))

    return out
